# baseline (speedup 1.0000x reference)
import functools

import jax
import jax.numpy as jnp
from jax import lax
from jax.experimental import pallas as pl
from jax.experimental.pallas import tpu as pltpu
from jax.experimental.pallas import tpu_sc as plsc


def _make_sc_gather(V, D, B, nc, ns):
    nw = nc * ns
    b_per_w = B // nw
    mesh = plsc.VectorSubcoreMesh(core_axis_name="c", subcore_axis_name="s")

    @functools.partial(
        pl.kernel,
        mesh=mesh,
        out_type=jax.ShapeDtypeStruct((B, D), jnp.float32),
        scratch_types=[
            pltpu.VMEM((b_per_w,), jnp.int32),
            pltpu.VMEM((b_per_w, D), jnp.float32),
            pltpu.SemaphoreType.DMA,
        ],
        compiler_params=pltpu.CompilerParams(use_tc_tiling_on_sc=False),
    )
    def gather(table_hbm, idx_hbm, out_hbm, idx_v, rows_v, sem):
        wid = lax.axis_index("s") * nc + lax.axis_index("c")
        base = wid * b_per_w
        pltpu.sync_copy(idx_hbm.at[pl.ds(base, b_per_w)], idx_v)
        pltpu.async_copy(table_hbm.at[idx_v], rows_v, sem).wait()
        pltpu.sync_copy(rows_v, out_hbm.at[pl.ds(base, b_per_w)])

    return gather


def _head_body(x_ref, w_ref, b_ref, o_ref):
    o_ref[...] = (
        jnp.dot(x_ref[...], w_ref[...], preferred_element_type=jnp.float32)
        + b_ref[...]
    )


def _head(x, head_w, head_b2d, bv):
    B, D = x.shape
    V = head_w.shape[1]
    return pl.pallas_call(
        _head_body,
        grid=(pl.cdiv(V, bv),),
        in_specs=[
            pl.BlockSpec((B, D), lambda j: (0, 0)),
            pl.BlockSpec((D, bv), lambda j: (0, j)),
            pl.BlockSpec((1, bv), lambda j: (0, j)),
        ],
        out_specs=pl.BlockSpec((B, bv), lambda j: (0, j)),
        out_shape=jax.ShapeDtypeStruct((B, V), jnp.float32),
    )(x, head_w, head_b2d)


def kernel(ids, embed_table, head_w, head_b):
    ids = ids.astype(jnp.int32)
    B = ids.shape[0]
    V, D = embed_table.shape
    info = plsc.get_sparse_core_info()
    x = _make_sc_gather(V, D, B, info.num_cores, info.num_subcores)(
        embed_table, ids
    )
    return _head(x, head_w, head_b.reshape(1, -1), 2048)


# speedup vs baseline: 2.1262x; 2.1262x over previous
import functools

import jax
import jax.numpy as jnp
from jax import lax
from jax.experimental import pallas as pl
from jax.experimental.pallas import tpu as pltpu
from jax.experimental.pallas import tpu_sc as plsc


def _make_sc_gather(V, D, B, nc, ns):
    nw = nc * ns
    b_per_w = B // nw
    mesh = plsc.VectorSubcoreMesh(core_axis_name="c", subcore_axis_name="s")

    @functools.partial(
        pl.kernel,
        mesh=mesh,
        out_type=jax.ShapeDtypeStruct((B, D), jnp.float32),
        scratch_types=[
            pltpu.VMEM((b_per_w,), jnp.int32),
            pltpu.VMEM((b_per_w, D), jnp.float32),
            pltpu.SemaphoreType.DMA,
        ],
        compiler_params=pltpu.CompilerParams(use_tc_tiling_on_sc=False),
    )
    def gather(table_hbm, idx_hbm, out_hbm, idx_v, rows_v, sem):
        wid = lax.axis_index("s") * nc + lax.axis_index("c")
        base = wid * b_per_w
        pltpu.sync_copy(idx_hbm.at[pl.ds(base, b_per_w)], idx_v)
        pltpu.async_copy(table_hbm.at[idx_v], rows_v, sem).wait()
        pltpu.sync_copy(rows_v, out_hbm.at[pl.ds(base, b_per_w)])

    return gather


def _head_t_body(w_ref, xt_ref, b_ref, o_ref):
    acc = jax.lax.dot_general(
        w_ref[...],
        xt_ref[...],
        dimension_numbers=(((0,), (0,)), ((), ())),
        preferred_element_type=jnp.float32,
    )
    o_ref[...] = acc + b_ref[...]


def _head_t(xt, head_w, head_b_col, bv):
    D, B = xt.shape
    V = head_w.shape[1]
    out_t = pl.pallas_call(
        _head_t_body,
        grid=(pl.cdiv(V, bv),),
        in_specs=[
            pl.BlockSpec((D, bv), lambda j: (0, j)),
            pl.BlockSpec((D, B), lambda j: (0, 0)),
            pl.BlockSpec((bv, 1), lambda j: (j, 0)),
        ],
        out_specs=pl.BlockSpec((bv, B), lambda j: (j, 0)),
        out_shape=jax.ShapeDtypeStruct((V, B), jnp.float32),
        compiler_params=pltpu.CompilerParams(
            fuse_transposed_lhs_in_matmul=True,
        ),
    )(head_w, xt, head_b_col)
    return out_t.T


def kernel(ids, embed_table, head_w, head_b):
    ids = ids.astype(jnp.int32)
    B = ids.shape[0]
    V, D = embed_table.shape
    info = plsc.get_sparse_core_info()
    x = _make_sc_gather(V, D, B, info.num_cores, info.num_subcores)(
        embed_table, ids
    )
    return _head_t(x.T, head_w, head_b.reshape(-1, 1), 2048)


# speedup vs baseline: 2.7586x; 1.2974x over previous
import functools

import jax
import jax.numpy as jnp
from jax import lax
from jax.experimental import pallas as pl
from jax.experimental.pallas import tpu as pltpu
from jax.experimental.pallas import tpu_sc as plsc


def _make_sc_gather(V, D, B, nc, ns):
    nw = nc * ns
    b_per_w = B // nw
    mesh = plsc.VectorSubcoreMesh(core_axis_name="c", subcore_axis_name="s")

    @functools.partial(
        pl.kernel,
        mesh=mesh,
        out_type=jax.ShapeDtypeStruct((B, D), jnp.float32),
        scratch_types=[
            pltpu.VMEM((b_per_w,), jnp.int32),
            pltpu.VMEM((b_per_w, D), jnp.float32),
            pltpu.SemaphoreType.DMA,
        ],
        compiler_params=pltpu.CompilerParams(use_tc_tiling_on_sc=False),
    )
    def gather(table_hbm, idx_hbm, out_hbm, idx_v, rows_v, sem):
        wid = lax.axis_index("s") * nc + lax.axis_index("c")
        base = wid * b_per_w
        pltpu.sync_copy(idx_hbm.at[pl.ds(base, b_per_w)], idx_v)
        pltpu.async_copy(table_hbm.at[idx_v], rows_v, sem).wait()
        pltpu.sync_copy(rows_v, out_hbm.at[pl.ds(base, b_per_w)])

    return gather


def _head_t_body(w_ref, xt_ref, b_ref, o_ref):
    wb = jnp.concatenate([w_ref[...], b_ref[...]], axis=0)
    ones = jnp.ones((1, xt_ref.shape[1]), jnp.float32)
    xt1 = jnp.concatenate([xt_ref[...], ones], axis=0)
    o_ref[...] = jax.lax.dot_general(
        wb,
        xt1,
        dimension_numbers=(((0,), (0,)), ((), ())),
        preferred_element_type=jnp.float32,
    )


def _head_t(xt, head_w, head_b_row, bv):
    D, B = xt.shape
    V = head_w.shape[1]
    out_t = pl.pallas_call(
        _head_t_body,
        grid=(pl.cdiv(V, bv),),
        in_specs=[
            pl.BlockSpec((D, bv), lambda j: (0, j)),
            pl.BlockSpec((D, B), lambda j: (0, 0)),
            pl.BlockSpec((1, bv), lambda j: (0, j)),
        ],
        out_specs=pl.BlockSpec((bv, B), lambda j: (j, 0)),
        out_shape=jax.ShapeDtypeStruct((V, B), jnp.float32),
        compiler_params=pltpu.CompilerParams(
            fuse_transposed_lhs_in_matmul=True,
        ),
    )(head_w, xt, head_b_row)
    return out_t.T


def kernel(ids, embed_table, head_w, head_b):
    ids = ids.astype(jnp.int32)
    B = ids.shape[0]
    V, D = embed_table.shape
    info = plsc.get_sparse_core_info()
    x = _make_sc_gather(V, D, B, info.num_cores, info.num_subcores)(
        embed_table, ids
    )
    return _head_t(x.T, head_w, head_b.reshape(1, -1), 2048)


# speedup vs baseline: 3.0695x; 1.1127x over previous
import functools

import jax
import jax.numpy as jnp
from jax import lax
from jax.experimental import pallas as pl
from jax.experimental.pallas import tpu as pltpu
from jax.experimental.pallas import tpu_sc as plsc


def _pack_body(t_ref, o_ref):
    t = jnp.transpose(t_ref[...])
    pad = jnp.zeros((t.shape[0], 128 - t.shape[1]), jnp.float32)
    o_ref[...] = jnp.concatenate([t, pad], axis=1)


def _pack(table_t, vpad, bg):
    D = table_t.shape[0]
    return pl.pallas_call(
        _pack_body,
        grid=(vpad // bg,),
        in_specs=[pl.BlockSpec((D, bg), lambda i: (0, i))],
        out_specs=pl.BlockSpec((bg, 128), lambda i: (i, 0)),
        out_shape=jax.ShapeDtypeStruct((vpad, 128), jnp.float32),
    )(table_t)


def _make_sc_gather(B, nc, ns):
    nw = nc * ns
    b_per_w = B // nw
    mesh = plsc.VectorSubcoreMesh(core_axis_name="c", subcore_axis_name="s")

    @functools.partial(
        pl.kernel,
        mesh=mesh,
        out_type=jax.ShapeDtypeStruct((B, 128), jnp.float32),
        scratch_types=[
            pltpu.VMEM((b_per_w,), jnp.int32),
            pltpu.VMEM((b_per_w, 128), jnp.float32),
            pltpu.SemaphoreType.DMA,
        ],
    )
    def gather(packed_hbm, idx_hbm, out_hbm, idx_v, rows_v, sem):
        wid = lax.axis_index("s") * nc + lax.axis_index("c")
        base = wid * b_per_w
        pltpu.sync_copy(idx_hbm.at[pl.ds(base, b_per_w)], idx_v)
        pltpu.async_copy(packed_hbm.at[idx_v], rows_v, sem).wait()
        pltpu.sync_copy(rows_v, out_hbm.at[pl.ds(base, b_per_w), :])

    return gather


def _head_t_body(w_ref, xt_ref, b_ref, o_ref):
    wb = jnp.concatenate([w_ref[...], b_ref[...]], axis=0)
    ones = jnp.ones((1, xt_ref.shape[1]), jnp.float32)
    xt1 = jnp.concatenate([xt_ref[...], ones], axis=0)
    o_ref[...] = jax.lax.dot_general(
        wb,
        xt1,
        dimension_numbers=(((0,), (0,)), ((), ())),
        preferred_element_type=jnp.float32,
    )


def _head_t(xt, head_w, head_b_row, bv):
    D, V = head_w.shape
    B = xt.shape[1]
    out_t = pl.pallas_call(
        _head_t_body,
        grid=(pl.cdiv(V, bv),),
        in_specs=[
            pl.BlockSpec((D, bv), lambda j: (0, j)),
            pl.BlockSpec((D, B), lambda j: (0, 0)),
            pl.BlockSpec((1, bv), lambda j: (0, j)),
        ],
        out_specs=pl.BlockSpec((bv, B), lambda j: (j, 0)),
        out_shape=jax.ShapeDtypeStruct((V, B), jnp.float32),
        compiler_params=pltpu.CompilerParams(
            fuse_transposed_lhs_in_matmul=True,
        ),
    )(head_w, xt, head_b_row)
    return out_t.T


def kernel(ids, embed_table, head_w, head_b):
    ids = ids.astype(jnp.int32)
    B = ids.shape[0]
    V, D = embed_table.shape
    vpad = 100096
    packed = _pack(embed_table.T, vpad, 4352)
    info = plsc.get_sparse_core_info()
    xpad = _make_sc_gather(B, info.num_cores, info.num_subcores)(packed, ids)
    return _head_t(xpad.T, head_w, head_b.reshape(1, -1), 2048)


# speedup vs baseline: 3.0759x; 1.0021x over previous
import functools

import jax
import jax.numpy as jnp
from jax import lax
from jax.experimental import pallas as pl
from jax.experimental.pallas import tpu as pltpu
from jax.experimental.pallas import tpu_sc as plsc


def _pack_body(t_ref, o_ref):
    t = jnp.transpose(t_ref[...])
    pad = jnp.zeros((t.shape[0], 128 - t.shape[1]), jnp.float32)
    o_ref[...] = jnp.concatenate([t, pad], axis=1)


def _pack(table_t, vpad, bg):
    D = table_t.shape[0]
    return pl.pallas_call(
        _pack_body,
        grid=(vpad // bg,),
        in_specs=[pl.BlockSpec((D, bg), lambda i: (0, i))],
        out_specs=pl.BlockSpec((bg, 128), lambda i: (i, 0)),
        out_shape=jax.ShapeDtypeStruct((vpad, 128), jnp.float32),
    )(table_t)


def _make_sc_gather(B, nc, ns):
    nw = nc * ns
    b_per_w = B // nw
    mesh = plsc.VectorSubcoreMesh(core_axis_name="c", subcore_axis_name="s")

    @functools.partial(
        pl.kernel,
        mesh=mesh,
        out_type=jax.ShapeDtypeStruct((B, 128), jnp.float32),
        scratch_types=[
            pltpu.VMEM((b_per_w,), jnp.int32),
            pltpu.VMEM((b_per_w, 128), jnp.float32),
            pltpu.SemaphoreType.DMA,
        ],
    )
    def gather(packed_hbm, idx_hbm, out_hbm, idx_v, rows_v, sem):
        wid = lax.axis_index("s") * nc + lax.axis_index("c")
        base = wid * b_per_w
        pltpu.sync_copy(idx_hbm.at[pl.ds(base, b_per_w)], idx_v)
        pltpu.async_copy(packed_hbm.at[idx_v], rows_v, sem).wait()
        pltpu.sync_copy(rows_v, out_hbm.at[pl.ds(base, b_per_w), :])

    return gather


def _head_t_body(w_ref, xt_ref, b_ref, o_ref):
    wb = jnp.concatenate([w_ref[...], b_ref[...]], axis=0)
    ones = jnp.ones((1, xt_ref.shape[1]), jnp.float32)
    xt1 = jnp.concatenate([xt_ref[...], ones], axis=0)
    o_ref[...] = jax.lax.dot_general(
        wb,
        xt1,
        dimension_numbers=(((0,), (0,)), ((), ())),
        preferred_element_type=jnp.float32,
    )


def _head_t(xt, head_w, head_b_row, bv):
    D, V = head_w.shape
    B = xt.shape[1]
    out_t = pl.pallas_call(
        _head_t_body,
        grid=(pl.cdiv(V, bv),),
        in_specs=[
            pl.BlockSpec((D, bv), lambda j: (0, j)),
            pl.BlockSpec((D, B), lambda j: (0, 0)),
            pl.BlockSpec((1, bv), lambda j: (0, j)),
        ],
        out_specs=pl.BlockSpec((bv, B), lambda j: (j, 0)),
        out_shape=jax.ShapeDtypeStruct((V, B), jnp.float32),
        compiler_params=pltpu.CompilerParams(
            fuse_transposed_lhs_in_matmul=True,
        ),
    )(head_w, xt, head_b_row)
    return out_t.T


def kernel(ids, embed_table, head_w, head_b):
    ids = ids.astype(jnp.int32)
    B = ids.shape[0]
    V, D = embed_table.shape
    vpad = 100096
    packed = _pack(embed_table.T, vpad, 4352)
    info = plsc.get_sparse_core_info()
    xpad = _make_sc_gather(B, info.num_cores, info.num_subcores)(packed, ids)
    return _head_t(xpad.T, head_w, head_b.reshape(1, -1), 4096)


# speedup vs baseline: 3.1450x; 1.0225x over previous
import functools

import jax
import jax.numpy as jnp
from jax import lax
from jax.experimental import pallas as pl
from jax.experimental.pallas import tpu as pltpu
from jax.experimental.pallas import tpu_sc as plsc


def _pack_body(t0, t1, t2, t3, o_ref):
    parts = [jnp.transpose(t[...]) for t in (t0, t1, t2, t3)]
    o_ref[...] = jnp.concatenate(parts, axis=1)


def _pack(table_t, g, bg):
    D = table_t.shape[0]
    nblk = g // bg

    def spec(j):
        return pl.BlockSpec((D, bg), lambda i, j=j: (0, i + nblk * j))

    return pl.pallas_call(
        _pack_body,
        grid=(nblk,),
        in_specs=[spec(0), spec(1), spec(2), spec(3)],
        out_specs=pl.BlockSpec((bg, 4 * D), lambda i: (i, 0)),
        out_shape=jax.ShapeDtypeStruct((g, 4 * D), jnp.float32),
    )(table_t, table_t, table_t, table_t)


def _make_sc_gather(G, B, nc, ns):
    nw = nc * ns
    b_per_w = B // nw
    mesh = plsc.VectorSubcoreMesh(core_axis_name="c", subcore_axis_name="s")

    @functools.partial(
        pl.kernel,
        mesh=mesh,
        out_type=jax.ShapeDtypeStruct((B, 128), jnp.float32),
        scratch_types=[
            pltpu.VMEM((b_per_w,), jnp.int32),
            pltpu.VMEM((b_per_w,), jnp.int32),
            pltpu.VMEM((b_per_w, 128), jnp.float32),
            pltpu.SemaphoreType.DMA,
        ],
    )
    def gather(packed_hbm, idx_hbm, out_hbm, idx_v, g_v, rows_v, sem):
        wid = lax.axis_index("s") * nc + lax.axis_index("c")
        base = wid * b_per_w
        pltpu.sync_copy(idx_hbm.at[pl.ds(base, b_per_w)], idx_v)
        for c in range(b_per_w // 16):
            v = idx_v[pl.ds(c * 16, 16)]
            g_v[pl.ds(c * 16, 16)] = v % G
        pltpu.async_copy(packed_hbm.at[g_v], rows_v, sem).wait()
        pltpu.sync_copy(rows_v, out_hbm.at[pl.ds(base, b_per_w), :])

    return gather


def _head_t_body(w_ref, xqt_ref, qsel_ref, b_ref, o_ref, xt_s):
    @pl.when(pl.program_id(0) == 0)
    def _():
        qsel = qsel_ref[...]
        acc = xqt_ref[0:32, :]
        for j in range(1, 4):
            acc = jnp.where(qsel == j, xqt_ref[32 * j:32 * j + 32, :], acc)
        xt_s[0:32, :] = acc
        xt_s[32:33, :] = jnp.ones((1, acc.shape[1]), jnp.float32)

    wb = jnp.concatenate([w_ref[...], b_ref[...]], axis=0)
    o_ref[...] = jax.lax.dot_general(
        wb,
        xt_s[...],
        dimension_numbers=(((0,), (0,)), ((), ())),
        preferred_element_type=jnp.float32,
    )


def _head_t(xqt, qsel_row, head_w, head_b_row, bv):
    D, V = head_w.shape
    B = xqt.shape[1]
    out_t = pl.pallas_call(
        _head_t_body,
        grid=(pl.cdiv(V, bv),),
        in_specs=[
            pl.BlockSpec((D, bv), lambda j: (0, j)),
            pl.BlockSpec((128, B), lambda j: (0, 0)),
            pl.BlockSpec((1, B), lambda j: (0, 0)),
            pl.BlockSpec((1, bv), lambda j: (0, j)),
        ],
        out_specs=pl.BlockSpec((bv, B), lambda j: (j, 0)),
        out_shape=jax.ShapeDtypeStruct((V, B), jnp.float32),
        scratch_shapes=[pltpu.VMEM((33, B), jnp.float32)],
        compiler_params=pltpu.CompilerParams(
            fuse_transposed_lhs_in_matmul=True,
        ),
    )(head_w, xqt, qsel_row, head_b_row)
    return out_t.T


def kernel(ids, embed_table, head_w, head_b):
    ids = ids.astype(jnp.int32)
    B = ids.shape[0]
    V, D = embed_table.shape
    g = 25088
    packed = _pack(embed_table.T, g, 3584)
    info = plsc.get_sparse_core_info()
    xq = _make_sc_gather(g, B, info.num_cores, info.num_subcores)(packed, ids)
    qsel_row = (ids // g).reshape(1, B)
    return _head_t(xq.T, qsel_row, head_w, head_b.reshape(1, -1), 4096)
